# Initial kernel scaffold; baseline (speedup 1.0000x reference)
import jax
import jax.numpy as jnp
from jax.experimental import pallas as pl


def kernel(x, edge_index, W, b):
    raise NotImplementedError("write your pallas kernel here")


# speedup vs baseline: 25.9581x; 25.9581x over previous
import functools

import jax
import jax.numpy as jnp
from jax import lax
from jax.experimental import pallas as pl
from jax.experimental.pallas import tpu as pltpu
from jax.experimental.pallas import tpu_sc as plsc

N = 10000
E = 320000
D = 128
NW = 32
NB = 80
K = 125
RPT = 624
ZC = 208

_mesh = plsc.VectorSubcoreMesh(core_axis_name="c", subcore_axis_name="s")


@functools.partial(
    pl.kernel,
    out_type=jax.ShapeDtypeStruct((2, N), jnp.float32),
    mesh=_mesh,
    scratch_types=[
        pltpu.VMEM_SHARED((N,), jnp.float32),
        pltpu.VMEM((NB, K), jnp.int32),
        pltpu.VMEM((128,), jnp.float32),
        pltpu.VMEM((1024,), jnp.float32),
    ],
)
def _deg_kernel(dst_hbm, pdeg_hbm, deg_s, dst_v, ones_v, zb_v):
    c = lax.axis_index("c")
    s = lax.axis_index("s")
    w = c * 16 + s
    pltpu.sync_copy(dst_hbm.at[w], dst_v)

    z16 = jnp.zeros((16,), jnp.float32)
    for k in range(8):
        ones_v[pl.ds(k * 16, 16)] = z16 + 1.0
    for k in range(64):
        zb_v[pl.ds(k * 16, 16)] = z16
    @pl.when(s < 10)
    def _():
        pltpu.sync_copy(zb_v.at[pl.ds(0, 1000)], deg_s.at[pl.ds(s * 1000, 1000)])

    plsc.subcore_barrier()

    def body(j, carry):
        pltpu.sync_copy(ones_v.at[pl.ds(0, K)], deg_s.at[dst_v.at[j]], add=True)
        return carry

    lax.fori_loop(0, NB, body, 0)
    plsc.subcore_barrier()

    @pl.when(s == 0)
    def _():
        pltpu.sync_copy(deg_s, pdeg_hbm.at[c])


@functools.partial(
    pl.kernel,
    out_type=jax.ShapeDtypeStruct((2, N, D), jnp.float32),
    mesh=_mesh,
    scratch_types=[
        pltpu.VMEM_SHARED((N, D), jnp.float32),
        pltpu.VMEM((NB, K), jnp.int32),
        pltpu.VMEM((NB, K), jnp.int32),
        pltpu.VMEM((ZC, D), jnp.float32),
        pltpu.SemaphoreType.DMA,
    ],
)
def _agg_kernel(xp_hbm, src_hbm, dst_hbm, pagg_hbm, agg_s, src_v, dst_v, buf_v, sem):
    c = lax.axis_index("c")
    s = lax.axis_index("s")
    w = c * 16 + s
    pltpu.sync_copy(src_hbm.at[w], src_v)
    pltpu.sync_copy(dst_hbm.at[w], dst_v)

    z16 = jnp.zeros((16,), jnp.float32)

    def zrow(i, carry):
        for k in range(8):
            buf_v[i, pl.ds(k * 16, 16)] = z16
        return carry

    lax.fori_loop(0, ZC, zrow, 0)
    base = s * RPT
    for k in range(3):
        pltpu.sync_copy(buf_v, agg_s.at[pl.ds(base + k * ZC, ZC)])

    @pl.when(s == 15)
    def _():
        pltpu.sync_copy(buf_v.at[pl.ds(0, 16)], agg_s.at[pl.ds(N - 16, 16)])

    plsc.subcore_barrier()

    def body(j, carry):
        pltpu.async_copy(xp_hbm.at[src_v.at[j]], buf_v.at[pl.ds(0, K)], sem).wait()
        pltpu.sync_copy(buf_v.at[pl.ds(0, K)], agg_s.at[dst_v.at[j]], add=True)
        return carry

    lax.fori_loop(0, NB, body, 0)
    plsc.subcore_barrier()

    pltpu.sync_copy(agg_s.at[pl.ds(base, RPT)], pagg_hbm.at[c, pl.ds(base, RPT)])

    @pl.when(s == 15)
    def _():
        pltpu.sync_copy(agg_s.at[pl.ds(N - 16, 16)], pagg_hbm.at[c, pl.ds(N - 16, 16)])


def _scale_body(pd_ref, x_ref, xp_ref, dis_ref):
    deg = pd_ref[:, 0] + pd_ref[:, 1]
    deg = jnp.maximum(deg, 1.0)
    dis = lax.rsqrt(deg)
    xp_ref[...] = x_ref[...] * dis[:, None]
    dis_ref[...] = dis[:, None]


_RB = 400


def _scale_call(pd2, x):
    return pl.pallas_call(
        _scale_body,
        grid=(N // _RB,),
        in_specs=[
            pl.BlockSpec((_RB, 2), lambda i: (i, 0)),
            pl.BlockSpec((_RB, D), lambda i: (i, 0)),
        ],
        out_specs=[
            pl.BlockSpec((_RB, D), lambda i: (i, 0)),
            pl.BlockSpec((_RB, 1), lambda i: (i, 0)),
        ],
        out_shape=[
            jax.ShapeDtypeStruct((N, D), jnp.float32),
            jax.ShapeDtypeStruct((N, 1), jnp.float32),
        ],
    )(pd2, x)


def _matmul_body(pagg_ref, dis_ref, w_ref, b_ref, out_ref):
    a = (pagg_ref[0] + pagg_ref[1]) * dis_ref[...]
    out_ref[...] = (
        jnp.dot(a, w_ref[...], preferred_element_type=jnp.float32) + b_ref[...]
    )


def _matmul_call(pagg, dis, W, b2):
    return pl.pallas_call(
        _matmul_body,
        grid=(N // _RB,),
        in_specs=[
            pl.BlockSpec((2, _RB, D), lambda i: (0, i, 0)),
            pl.BlockSpec((_RB, 1), lambda i: (i, 0)),
            pl.BlockSpec((D, D), lambda i: (0, 0)),
            pl.BlockSpec((1, D), lambda i: (0, 0)),
        ],
        out_specs=pl.BlockSpec((_RB, D), lambda i: (i, 0)),
        out_shape=jax.ShapeDtypeStruct((N, D), jnp.float32),
    )(pagg, dis, W, b2)


def kernel(x, edge_index, W, b):
    src3 = edge_index[0].reshape(NW, NB, K)
    dst3 = edge_index[1].reshape(NW, NB, K)
    pdeg = _deg_kernel(dst3)
    xp, dis = _scale_call(pdeg.T, x)
    pagg = _agg_kernel(xp, src3, dst3)
    return _matmul_call(pagg, dis, W, b.reshape(1, D))


# speedup vs baseline: 30.1001x; 1.1596x over previous
import functools

import jax
import jax.numpy as jnp
from jax import lax
from jax.experimental import pallas as pl
from jax.experimental.pallas import tpu as pltpu
from jax.experimental.pallas import tpu_sc as plsc

N = 10000
E = 320000
D = 128
NW = 32
NB = 80
KR = 125
K = 128
NPAD = 240
NP = N + NPAD
RPT = 624

_mesh = plsc.VectorSubcoreMesh(core_axis_name="c", subcore_axis_name="s")


@functools.partial(
    pl.kernel,
    out_type=jax.ShapeDtypeStruct((2, N), jnp.float32),
    mesh=_mesh,
    scratch_types=[
        pltpu.VMEM_SHARED((N,), jnp.float32),
        pltpu.VMEM((NB, K), jnp.int32),
        pltpu.VMEM((K,), jnp.float32),
        pltpu.VMEM((1024,), jnp.float32),
    ],
)
def _deg_kernel(dst_hbm, pdeg_hbm, deg_s, dst_v, ones_v, zb_v):
    c = lax.axis_index("c")
    s = lax.axis_index("s")
    w = c * 16 + s
    pltpu.sync_copy(dst_hbm.at[w], dst_v)

    z16 = jnp.zeros((16,), jnp.float32)
    for k in range(7):
        ones_v[pl.ds(k * 16, 16)] = z16 + 1.0
    lane = lax.iota(jnp.int32, 16)
    ones_v[pl.ds(112, 16)] = jnp.where(lane < 13, 1.0, 0.0).astype(jnp.float32)
    for k in range(64):
        zb_v[pl.ds(k * 16, 16)] = z16
    @pl.when(s < 10)
    def _():
        pltpu.sync_copy(zb_v.at[pl.ds(0, 1000)], deg_s.at[pl.ds(s * 1000, 1000)])

    plsc.subcore_barrier()

    def body(j, carry):
        pltpu.sync_copy(ones_v, deg_s.at[dst_v.at[j]], add=True)
        return carry

    lax.fori_loop(0, NB, body, 0)
    plsc.subcore_barrier()

    @pl.when(s == 0)
    def _():
        pltpu.sync_copy(deg_s, pdeg_hbm.at[c])


@functools.partial(
    pl.kernel,
    out_type=jax.ShapeDtypeStruct((2, N, D), jnp.float32),
    mesh=_mesh,
    scratch_types=[
        pltpu.VMEM_SHARED((N, D), jnp.float32),
        pltpu.VMEM((2, K), jnp.int32),
        pltpu.VMEM((2, K), jnp.int32),
        pltpu.VMEM((K, D), jnp.float32),
        pltpu.VMEM((K, D), jnp.float32),
        pltpu.SemaphoreType.DMA,
        pltpu.SemaphoreType.DMA,
        pltpu.SemaphoreType.DMA,
        pltpu.SemaphoreType.DMA,
    ],
)
def _agg_kernel(xp_hbm, src_hbm, dst_hbm, pagg_hbm, agg_s, sidx, didx,
                buf_0, buf_1, sem_0, sem_1, sem_i0, sem_i1):
    c = lax.axis_index("c")
    s = lax.axis_index("s")
    w = c * 16 + s

    z16 = jnp.zeros((16,), jnp.float32)

    def zrow(i, carry):
        for k in range(8):
            buf_0[i, pl.ds(k * 16, 16)] = z16
        return carry

    lax.fori_loop(0, 104, zrow, 0)
    base = s * RPT
    for k in range(6):
        pltpu.sync_copy(buf_0.at[pl.ds(0, 104)], agg_s.at[pl.ds(base + k * 104, 104)])

    @pl.when(s == 15)
    def _():
        pltpu.sync_copy(buf_0.at[pl.ds(0, 16)], agg_s.at[pl.ds(N - 16, 16)])

    plsc.subcore_barrier()

    bufs = (buf_0, buf_1)
    sems = (sem_0, sem_1)
    isems = (sem_i0, sem_i1)

    pltpu.sync_copy(src_hbm.at[w, 0], sidx.at[0])
    pltpu.sync_copy(dst_hbm.at[w, 0], didx.at[0])
    pltpu.async_copy(src_hbm.at[w, 1], sidx.at[1], sem_i1)
    pltpu.async_copy(dst_hbm.at[w, 1], didx.at[1], sem_i1)
    pltpu.async_copy(xp_hbm.at[sidx.at[0]], buf_0, sem_0)

    def body(i, carry):
        j2 = i * 2
        for p in range(2):
            j = j2 + p
            q = 1 - p
            buf, sem = bufs[p], sems[p]
            pltpu.make_async_copy(xp_hbm.at[sidx.at[p]], buf, sem).wait()

            @pl.when(j + 1 < NB)
            def _():
                pltpu.make_async_copy(src_hbm.at[w, j + 1], sidx.at[q],
                                      isems[q]).wait()
                pltpu.make_async_copy(dst_hbm.at[w, j + 1], didx.at[q],
                                      isems[q]).wait()
                pltpu.async_copy(xp_hbm.at[sidx.at[q]], bufs[q], sems[q])

            pltpu.sync_copy(buf, agg_s.at[didx.at[p]], add=True)

            @pl.when(j + 2 < NB)
            def _():
                pltpu.async_copy(src_hbm.at[w, j + 2], sidx.at[p], isems[p])
                pltpu.async_copy(dst_hbm.at[w, j + 2], didx.at[p], isems[p])
        return carry

    lax.fori_loop(0, NB // 2, body, 0)
    plsc.subcore_barrier()

    pltpu.sync_copy(agg_s.at[pl.ds(base, RPT)], pagg_hbm.at[c, pl.ds(base, RPT)])

    @pl.when(s == 15)
    def _():
        pltpu.sync_copy(agg_s.at[pl.ds(N - 16, 16)], pagg_hbm.at[c, pl.ds(N - 16, 16)])


def _scale_body(pd_ref, x_ref, xp_ref, dis_ref):
    deg = pd_ref[:, 0] + pd_ref[:, 1]
    deg = jnp.maximum(deg, 1.0)
    dis = lax.rsqrt(deg)
    xp_ref[...] = x_ref[...] * dis[:, None]
    dis_ref[...] = dis[:, None]


_RBB = 512


def _scale_call(pd2, xpad):
    return pl.pallas_call(
        _scale_body,
        grid=(NP // _RBB,),
        in_specs=[
            pl.BlockSpec((_RBB, 2), lambda i: (i, 0)),
            pl.BlockSpec((_RBB, D), lambda i: (i, 0)),
        ],
        out_specs=[
            pl.BlockSpec((_RBB, D), lambda i: (i, 0)),
            pl.BlockSpec((_RBB, 1), lambda i: (i, 0)),
        ],
        out_shape=[
            jax.ShapeDtypeStruct((NP, D), jnp.float32),
            jax.ShapeDtypeStruct((NP, 1), jnp.float32),
        ],
    )(pd2, xpad)


def _matmul_body(pagg_ref, dis_ref, w_ref, b_ref, out_ref):
    a = (pagg_ref[0] + pagg_ref[1]) * dis_ref[...]
    out_ref[...] = (
        jnp.dot(a, w_ref[...], preferred_element_type=jnp.float32) + b_ref[...]
    )


_RB = 400


def _matmul_call(pagg, dis, W, b2):
    return pl.pallas_call(
        _matmul_body,
        grid=(N // _RB,),
        in_specs=[
            pl.BlockSpec((2, _RB, D), lambda i: (0, i, 0)),
            pl.BlockSpec((_RB, 1), lambda i: (i, 0)),
            pl.BlockSpec((D, D), lambda i: (0, 0)),
            pl.BlockSpec((1, D), lambda i: (0, 0)),
        ],
        out_specs=pl.BlockSpec((_RB, D), lambda i: (i, 0)),
        out_shape=jax.ShapeDtypeStruct((N, D), jnp.float32),
    )(pagg, dis, W, b2)


def kernel(x, edge_index, W, b):
    src = edge_index[0].reshape(NW, NB, KR)
    dst = edge_index[1].reshape(NW, NB, KR)
    spread = jnp.arange(NW * NB * 3, dtype=jnp.int32).reshape(NW, NB, 3)
    src_p = jnp.concatenate([src, N + spread % NPAD], axis=2)
    dst_p = jnp.concatenate([dst, (spread * 37) % N], axis=2)

    pdeg = _deg_kernel(dst_p)
    pd2 = jnp.pad(pdeg.T, ((0, NPAD), (0, 0)), constant_values=1.0)
    xpad = jnp.pad(x, ((0, NPAD), (0, 0)))
    xp, dis = _scale_call(pd2, xpad)
    pagg = _agg_kernel(xp, src_p, dst_p)
    return _matmul_call(pagg, dis[:N], W, b.reshape(1, D))


# speedup vs baseline: 31.6667x; 1.0520x over previous
import functools

import jax
import jax.numpy as jnp
from jax import lax
from jax.experimental import pallas as pl
from jax.experimental.pallas import tpu as pltpu
from jax.experimental.pallas import tpu_sc as plsc

N = 10000
E = 320000
D = 128
NW = 32
NB = 81
KR = 125
K = 128
NPAD = 240
NP = N + NPAD
RPT = 624

_mesh = plsc.VectorSubcoreMesh(core_axis_name="c", subcore_axis_name="s")


@functools.partial(
    pl.kernel,
    out_type=jax.ShapeDtypeStruct((2, N), jnp.float32),
    mesh=_mesh,
    scratch_types=[
        pltpu.VMEM_SHARED((N,), jnp.float32),
        pltpu.VMEM((NB, K), jnp.int32),
        pltpu.VMEM((K,), jnp.float32),
        pltpu.VMEM((1024,), jnp.float32),
    ],
)
def _deg_kernel(dst_hbm, pdeg_hbm, deg_s, dst_v, ones_v, zb_v):
    c = lax.axis_index("c")
    s = lax.axis_index("s")
    w = c * 16 + s
    pltpu.sync_copy(dst_hbm.at[w], dst_v)

    z16 = jnp.zeros((16,), jnp.float32)
    for k in range(7):
        ones_v[pl.ds(k * 16, 16)] = z16 + 1.0
    lane = lax.iota(jnp.int32, 16)
    ones_v[pl.ds(112, 16)] = jnp.where(lane < 13, 1.0, 0.0).astype(jnp.float32)
    for k in range(64):
        zb_v[pl.ds(k * 16, 16)] = z16
    @pl.when(s < 10)
    def _():
        pltpu.sync_copy(zb_v.at[pl.ds(0, 1000)], deg_s.at[pl.ds(s * 1000, 1000)])

    plsc.subcore_barrier()

    def body(j, carry):
        pltpu.sync_copy(ones_v, deg_s.at[dst_v.at[j]], add=True)
        return carry

    lax.fori_loop(0, NB - 1, body, 0)
    plsc.subcore_barrier()

    @pl.when(s == 0)
    def _():
        pltpu.sync_copy(deg_s, pdeg_hbm.at[c])


@functools.partial(
    pl.kernel,
    out_type=jax.ShapeDtypeStruct((2, N, D), jnp.float32),
    mesh=_mesh,
    scratch_types=[
        pltpu.VMEM_SHARED((N, D), jnp.float32),
        pltpu.VMEM((3, K), jnp.int32),
        pltpu.VMEM((3, K), jnp.int32),
        pltpu.VMEM((K, D), jnp.float32),
        pltpu.VMEM((K, D), jnp.float32),
        pltpu.VMEM((K, D), jnp.float32),
        pltpu.SemaphoreType.DMA,
        pltpu.SemaphoreType.DMA,
        pltpu.SemaphoreType.DMA,
        pltpu.SemaphoreType.DMA,
        pltpu.SemaphoreType.DMA,
        pltpu.SemaphoreType.DMA,
    ],
)
def _agg_kernel(xp_hbm, src_hbm, dst_hbm, pagg_hbm, agg_s, sidx, didx,
                buf_0, buf_1, buf_2, sem_0, sem_1, sem_2,
                sem_i0, sem_i1, sem_i2):
    c = lax.axis_index("c")
    s = lax.axis_index("s")
    w = c * 16 + s

    z16 = jnp.zeros((16,), jnp.float32)

    def zrow(i, carry):
        for k in range(8):
            buf_0[i, pl.ds(k * 16, 16)] = z16
        return carry

    lax.fori_loop(0, 104, zrow, 0)
    base = s * RPT
    for k in range(6):
        pltpu.sync_copy(buf_0.at[pl.ds(0, 104)], agg_s.at[pl.ds(base + k * 104, 104)])

    @pl.when(s == 15)
    def _():
        pltpu.sync_copy(buf_0.at[pl.ds(0, 16)], agg_s.at[pl.ds(N - 16, 16)])

    plsc.subcore_barrier()

    bufs = (buf_0, buf_1, buf_2)
    sems = (sem_0, sem_1, sem_2)
    isems = (sem_i0, sem_i1, sem_i2)

    pltpu.sync_copy(src_hbm.at[w, 0], sidx.at[0])
    pltpu.sync_copy(dst_hbm.at[w, 0], didx.at[0])
    pltpu.sync_copy(src_hbm.at[w, 1], sidx.at[1])
    pltpu.sync_copy(dst_hbm.at[w, 1], didx.at[1])
    pltpu.async_copy(xp_hbm.at[sidx.at[0]], buf_0, sem_0)
    pltpu.async_copy(xp_hbm.at[sidx.at[1]], buf_1, sem_1)
    pltpu.async_copy(src_hbm.at[w, 2], sidx.at[2], sem_i2)
    pltpu.async_copy(dst_hbm.at[w, 2], didx.at[2], sem_i2)

    def body(i, carry):
        j3 = i * 3
        for p in range(3):
            j = j3 + p
            r = (p + 2) % 3
            pltpu.make_async_copy(xp_hbm.at[sidx.at[p]], bufs[p], sems[p]).wait()

            @pl.when(j + 2 < NB)
            def _():
                pltpu.make_async_copy(src_hbm.at[w, j + 2], sidx.at[r],
                                      isems[r]).wait()
                pltpu.make_async_copy(dst_hbm.at[w, j + 2], didx.at[r],
                                      isems[r]).wait()
                pltpu.async_copy(xp_hbm.at[sidx.at[r]], bufs[r], sems[r])

            pltpu.sync_copy(bufs[p], agg_s.at[didx.at[p]], add=True)

            @pl.when(j + 3 < NB)
            def _():
                pltpu.async_copy(src_hbm.at[w, j + 3], sidx.at[p], isems[p])
                pltpu.async_copy(dst_hbm.at[w, j + 3], didx.at[p], isems[p])
        return carry

    lax.fori_loop(0, NB // 3, body, 0)
    plsc.subcore_barrier()

    pltpu.sync_copy(agg_s.at[pl.ds(base, RPT)], pagg_hbm.at[c, pl.ds(base, RPT)])

    @pl.when(s == 15)
    def _():
        pltpu.sync_copy(agg_s.at[pl.ds(N - 16, 16)], pagg_hbm.at[c, pl.ds(N - 16, 16)])


def _scale_body(pd_ref, x_ref, xp_ref, dis_ref):
    deg = pd_ref[:, 0] + pd_ref[:, 1]
    deg = jnp.maximum(deg, 1.0)
    dis = lax.rsqrt(deg)
    xp_ref[...] = x_ref[...] * dis[:, None]
    dis_ref[...] = dis[:, None]


_RBB = 512


def _scale_call(pd2, xpad):
    return pl.pallas_call(
        _scale_body,
        grid=(NP // _RBB,),
        in_specs=[
            pl.BlockSpec((_RBB, 2), lambda i: (i, 0)),
            pl.BlockSpec((_RBB, D), lambda i: (i, 0)),
        ],
        out_specs=[
            pl.BlockSpec((_RBB, D), lambda i: (i, 0)),
            pl.BlockSpec((_RBB, 1), lambda i: (i, 0)),
        ],
        out_shape=[
            jax.ShapeDtypeStruct((NP, D), jnp.float32),
            jax.ShapeDtypeStruct((NP, 1), jnp.float32),
        ],
    )(pd2, xpad)


def _matmul_body(pagg_ref, dis_ref, w_ref, b_ref, out_ref):
    a = (pagg_ref[0] + pagg_ref[1]) * dis_ref[...]
    out_ref[...] = (
        jnp.dot(a, w_ref[...], preferred_element_type=jnp.float32) + b_ref[...]
    )


_RB = 400


def _matmul_call(pagg, dis, W, b2):
    return pl.pallas_call(
        _matmul_body,
        grid=(N // _RB,),
        in_specs=[
            pl.BlockSpec((2, _RB, D), lambda i: (0, i, 0)),
            pl.BlockSpec((_RB, 1), lambda i: (i, 0)),
            pl.BlockSpec((D, D), lambda i: (0, 0)),
            pl.BlockSpec((1, D), lambda i: (0, 0)),
        ],
        out_specs=pl.BlockSpec((_RB, D), lambda i: (i, 0)),
        out_shape=jax.ShapeDtypeStruct((N, D), jnp.float32),
    )(pagg, dis, W, b2)


def kernel(x, edge_index, W, b):
    src = edge_index[0].reshape(NW, NB - 1, KR)
    dst = edge_index[1].reshape(NW, NB - 1, KR)
    spread = jnp.arange(NW * (NB - 1) * 3, dtype=jnp.int32).reshape(NW, NB - 1, 3)
    src_p = jnp.concatenate([src, N + spread % NPAD], axis=2)
    dst_p = jnp.concatenate([dst, (spread * 37) % N], axis=2)
    tail = jnp.arange(NW * K, dtype=jnp.int32).reshape(NW, 1, K)
    src_p = jnp.concatenate([src_p, N + tail % NPAD], axis=1)
    dst_p = jnp.concatenate([dst_p, (tail * 37) % N], axis=1)

    pdeg = _deg_kernel(dst_p)
    pd2 = jnp.pad(pdeg.T, ((0, NPAD), (0, 0)), constant_values=1.0)
    xpad = jnp.pad(x, ((0, NPAD), (0, 0)))
    xp, dis = _scale_call(pd2, xpad)
    pagg = _agg_kernel(xp, src_p, dst_p)
    return _matmul_call(pagg, dis[:N], W, b.reshape(1, D))


# speedup vs baseline: 31.6721x; 1.0002x over previous
import functools

import jax
import jax.numpy as jnp
from jax import lax
from jax.experimental import pallas as pl
from jax.experimental.pallas import tpu as pltpu
from jax.experimental.pallas import tpu_sc as plsc

N = 10000
E = 320000
D = 128
NW = 32
NB = 81
KR = 125
K = 128
NPAD = 240
NP = N + NPAD
RPT = 624

_mesh = plsc.VectorSubcoreMesh(core_axis_name="c", subcore_axis_name="s")


@functools.partial(
    pl.kernel,
    out_type=jax.ShapeDtypeStruct((2, N), jnp.float32),
    mesh=_mesh,
    scratch_types=[
        pltpu.VMEM_SHARED((N,), jnp.float32),
        pltpu.VMEM((NB, K), jnp.int32),
        pltpu.VMEM((K,), jnp.float32),
        pltpu.VMEM((1024,), jnp.float32),
    ],
)
def _deg_kernel(dst_hbm, pdeg_hbm, deg_s, dst_v, ones_v, zb_v):
    c = lax.axis_index("c")
    s = lax.axis_index("s")
    w = c * 16 + s
    pltpu.sync_copy(dst_hbm.at[w], dst_v)

    z16 = jnp.zeros((16,), jnp.float32)
    for k in range(7):
        ones_v[pl.ds(k * 16, 16)] = z16 + 1.0
    lane = lax.iota(jnp.int32, 16)
    ones_v[pl.ds(112, 16)] = jnp.where(lane < 13, 1.0, 0.0).astype(jnp.float32)
    for k in range(64):
        zb_v[pl.ds(k * 16, 16)] = z16
    @pl.when(s < 10)
    def _():
        pltpu.sync_copy(zb_v.at[pl.ds(0, 1000)], deg_s.at[pl.ds(s * 1000, 1000)])

    plsc.subcore_barrier()

    def body(j, carry):
        pltpu.sync_copy(ones_v, deg_s.at[dst_v.at[j]], add=True)
        return carry

    lax.fori_loop(0, NB - 1, body, 0)
    plsc.subcore_barrier()

    @pl.when(s == 0)
    def _():
        pltpu.sync_copy(deg_s, pdeg_hbm.at[c])


@functools.partial(
    pl.kernel,
    out_type=jax.ShapeDtypeStruct((2, N, D), jnp.float32),
    mesh=_mesh,
    scratch_types=[
        pltpu.VMEM_SHARED((N, D), jnp.float32),
        pltpu.VMEM((3, K), jnp.int32),
        pltpu.VMEM((3, K), jnp.int32),
        pltpu.VMEM((K, D), jnp.float32),
        pltpu.VMEM((K, D), jnp.float32),
        pltpu.VMEM((K, D), jnp.float32),
        pltpu.SemaphoreType.DMA,
        pltpu.SemaphoreType.DMA,
        pltpu.SemaphoreType.DMA,
        pltpu.SemaphoreType.DMA,
        pltpu.SemaphoreType.DMA,
        pltpu.SemaphoreType.DMA,
    ],
)
def _agg_kernel(xp_hbm, src_hbm, dst_hbm, pagg_hbm, agg_s, sidx, didx,
                buf_0, buf_1, buf_2, sem_0, sem_1, sem_2,
                sem_i0, sem_i1, sem_i2):
    c = lax.axis_index("c")
    s = lax.axis_index("s")
    w = c * 16 + s

    z16 = jnp.zeros((16,), jnp.float32)

    def zrow(i, carry):
        for k in range(8):
            buf_0[i, pl.ds(k * 16, 16)] = z16
        return carry

    lax.fori_loop(0, 104, zrow, 0)
    base = s * RPT
    for k in range(6):
        pltpu.sync_copy(buf_0.at[pl.ds(0, 104)], agg_s.at[pl.ds(base + k * 104, 104)])

    @pl.when(s == 15)
    def _():
        pltpu.sync_copy(buf_0.at[pl.ds(0, 16)], agg_s.at[pl.ds(N - 16, 16)])

    plsc.subcore_barrier()

    bufs = (buf_0, buf_1, buf_2)
    sems = (sem_0, sem_1, sem_2)
    isems = (sem_i0, sem_i1, sem_i2)

    pltpu.sync_copy(src_hbm.at[w, 0], sidx.at[0])
    pltpu.sync_copy(dst_hbm.at[w, 0], didx.at[0])
    pltpu.sync_copy(src_hbm.at[w, 1], sidx.at[1])
    pltpu.sync_copy(dst_hbm.at[w, 1], didx.at[1])
    pltpu.async_copy(xp_hbm.at[sidx.at[0]], buf_0, sem_0)
    pltpu.async_copy(xp_hbm.at[sidx.at[1]], buf_1, sem_1)
    pltpu.async_copy(src_hbm.at[w, 2], sidx.at[2], sem_i2)
    pltpu.async_copy(dst_hbm.at[w, 2], didx.at[2], sem_i2)

    def body(i, carry):
        j3 = i * 3
        for p in range(3):
            j = j3 + p
            r = (p + 2) % 3
            pltpu.make_async_copy(xp_hbm.at[sidx.at[p]], bufs[p], sems[p]).wait()

            @pl.when(j + 2 < NB)
            def _():
                pltpu.make_async_copy(src_hbm.at[w, j + 2], sidx.at[r],
                                      isems[r]).wait()
                pltpu.make_async_copy(dst_hbm.at[w, j + 2], didx.at[r],
                                      isems[r]).wait()
                pltpu.async_copy(xp_hbm.at[sidx.at[r]], bufs[r], sems[r])

            pltpu.sync_copy(bufs[p], agg_s.at[didx.at[p]], add=True)

            @pl.when(j + 3 < NB)
            def _():
                pltpu.async_copy(src_hbm.at[w, j + 3], sidx.at[p], isems[p])
                pltpu.async_copy(dst_hbm.at[w, j + 3], didx.at[p], isems[p])
        return carry

    lax.fori_loop(0, NB // 3, body, 0)
    plsc.subcore_barrier()

    pltpu.sync_copy(agg_s.at[pl.ds(base, RPT)], pagg_hbm.at[c, pl.ds(base, RPT)])

    @pl.when(s == 15)
    def _():
        pltpu.sync_copy(agg_s.at[pl.ds(N - 16, 16)], pagg_hbm.at[c, pl.ds(N - 16, 16)])


_RBB = 512


def _scale_body(pd0_ref, pd1_ref, x_ref, xp_ref, dis_ref):
    i = pl.program_id(0)
    deg = pd0_ref[0, :, 0] + pd1_ref[0, :, 0]
    deg = jnp.maximum(deg, 1.0)
    dis = lax.rsqrt(deg)[:, None]
    row = i * _RBB + lax.broadcasted_iota(jnp.int32, (_RBB, 1), 0)
    pad = row >= N
    xp_ref[...] = jnp.where(pad, 0.0, x_ref[...] * dis)
    dis_ref[...] = jnp.where(pad, 1.0, dis)


def _scale_call(pdeg3, x):
    return pl.pallas_call(
        _scale_body,
        grid=(NP // _RBB,),
        in_specs=[
            pl.BlockSpec((1, _RBB, 1), lambda i: (0, i, 0)),
            pl.BlockSpec((1, _RBB, 1), lambda i: (1, i, 0)),
            pl.BlockSpec((_RBB, D), lambda i: (i, 0)),
        ],
        out_specs=[
            pl.BlockSpec((_RBB, D), lambda i: (i, 0)),
            pl.BlockSpec((_RBB, 1), lambda i: (i, 0)),
        ],
        out_shape=[
            jax.ShapeDtypeStruct((NP, D), jnp.float32),
            jax.ShapeDtypeStruct((NP, 1), jnp.float32),
        ],
    )(pdeg3, pdeg3, x)


def _matmul_body(pagg_ref, dis_ref, w_ref, b_ref, out_ref):
    a = (pagg_ref[0] + pagg_ref[1]) * dis_ref[...]
    out_ref[...] = (
        jnp.dot(a, w_ref[...], preferred_element_type=jnp.float32) + b_ref[...]
    )


_RB = 400


def _matmul_call(pagg, dis, W, b2):
    return pl.pallas_call(
        _matmul_body,
        grid=(N // _RB,),
        in_specs=[
            pl.BlockSpec((2, _RB, D), lambda i: (0, i, 0)),
            pl.BlockSpec((_RB, 1), lambda i: (i, 0)),
            pl.BlockSpec((D, D), lambda i: (0, 0)),
            pl.BlockSpec((1, D), lambda i: (0, 0)),
        ],
        out_specs=pl.BlockSpec((_RB, D), lambda i: (i, 0)),
        out_shape=jax.ShapeDtypeStruct((N, D), jnp.float32),
    )(pagg, dis, W, b2)


def kernel(x, edge_index, W, b):
    src = edge_index[0].reshape(NW, NB - 1, KR)
    dst = edge_index[1].reshape(NW, NB - 1, KR)
    spread = jnp.arange(NW * (NB - 1) * 3, dtype=jnp.int32).reshape(NW, NB - 1, 3)
    src_p = jnp.concatenate([src, N + spread % NPAD], axis=2)
    dst_p = jnp.concatenate([dst, (spread * 37) % N], axis=2)
    tail = jnp.arange(NW * K, dtype=jnp.int32).reshape(NW, 1, K)
    src_p = jnp.concatenate([src_p, N + tail % NPAD], axis=1)
    dst_p = jnp.concatenate([dst_p, (tail * 37) % N], axis=1)

    pdeg = _deg_kernel(dst_p)
    xp, dis = _scale_call(pdeg.reshape(2, N, 1), x)
    pagg = _agg_kernel(xp, src_p, dst_p)
    return _matmul_call(pagg, dis, W, b.reshape(1, D))
